# baseline (speedup 1.0000x reference)
import functools

import jax
import jax.numpy as jnp
from jax import lax
from jax.experimental import pallas as pl
from jax.experimental.pallas import tpu as pltpu
from jax.experimental.pallas import tpu_sc as plsc

N, E, D = 10000, 320000, 128
NC, NS = 2, 16
NW = NC * NS
NH = N // NC
CH = 128
VR = 16
EPW_A = ((E // NS + CH - 1) // CH) * CH
NCHK_A = EPW_A // CH
EPW_D = ((E // NW + CH - 1) // CH) * CH
NCHK_D = EPW_D // CH
CAP = EPW_D + CH
CAPC = CAP // CH
NPH = 5248
RPH = NPH // NS
NDUM = 240
NPAD = 10240
RPS = NPAD // NS


def _make_sc_agg():
    mesh = plsc.VectorSubcoreMesh(core_axis_name="c", subcore_axis_name="s")
    scratch = [
        pltpu.VMEM((NCHK_A, CH), jnp.int32),
        pltpu.VMEM((NCHK_A, CH), jnp.int32),
        pltpu.VMEM((CH,), jnp.int32),
        pltpu.VMEM((CH, D), jnp.float32),
        pltpu.VMEM((CH, D), jnp.float32),
        pltpu.VMEM_SHARED((NPH, D), jnp.float32),
        pltpu.SemaphoreType.DMA,
        pltpu.SemaphoreType.DMA,
    ]

    def body(h_hbm, srcw, dstw, zrow, out_p, src_v, dst_v, dloc_v, rows_a,
             rows_b, acc_sh, sem_a, sem_b):
        c = lax.axis_index("c")
        s = lax.axis_index("s")
        r0 = s * RPH
        lo = c * NH
        pltpu.sync_copy(srcw.at[s], src_v)
        pltpu.sync_copy(dstw.at[s], dst_v)
        pltpu.sync_copy(zrow.at[pl.ds(r0, RPH)], acc_sh.at[pl.ds(r0, RPH)])
        plsc.subcore_barrier()

        def remap(j):
            for k in range(CH // VR):
                d = dst_v[j, pl.ds(k * VR, VR)]
                dl = d - lo
                mine = (dl >= 0) & (dl < NH)
                dummy = NH + (j * (CH // VR) + k) % NDUM
                dloc_v[pl.ds(k * VR, VR)] = jnp.where(mine, dl, dummy)

        pltpu.async_copy(h_hbm.at[src_v.at[0]], rows_a, sem_a)

        def chunk(j, _):
            even = lax.rem(j, 2) == 0

            def do(rows_cur, sem_cur, rows_nxt, sem_nxt):
                pltpu.async_copy(h_hbm.at[src_v.at[j + 1]], rows_nxt, sem_nxt)
                remap(j)
                pltpu.make_async_copy(h_hbm.at[src_v.at[j]], rows_cur,
                                      sem_cur).wait()
                pltpu.sync_copy(rows_cur, acc_sh.at[dloc_v], add=True)

            lax.cond(even,
                     lambda: do(rows_a, sem_a, rows_b, sem_b),
                     lambda: do(rows_b, sem_b, rows_a, sem_a))
            return 0

        lax.fori_loop(0, NCHK_A - 1, chunk, 0, unroll=False)
        j = NCHK_A - 1
        rows_cur, sem_cur = (rows_a, sem_a) if j % 2 == 0 else (rows_b, sem_b)
        remap(j)
        pltpu.make_async_copy(h_hbm.at[src_v.at[j]], rows_cur, sem_cur).wait()
        pltpu.sync_copy(rows_cur, acc_sh.at[dloc_v], add=True)
        plsc.subcore_barrier()
        pltpu.sync_copy(acc_sh.at[pl.ds(r0, RPH)], out_p.at[c, pl.ds(r0, RPH)])

    return pl.kernel(body,
                     out_type=jax.ShapeDtypeStruct((NC, NPH, D), jnp.float32),
                     mesh=mesh, scratch_types=scratch, name="sc_segsum")


def _make_sc_part():
    mesh = plsc.VectorSubcoreMesh(core_axis_name="c", subcore_axis_name="s")
    out_type = (jax.ShapeDtypeStruct((NC, NW, CAP), jnp.int32),
                jax.ShapeDtypeStruct((NC, NW, CAP), jnp.int32),
                jax.ShapeDtypeStruct((NC, NW, CH), jnp.int32))
    scratch = [
        pltpu.VMEM((NCHK_D, CH), jnp.int32),
        pltpu.VMEM((NCHK_D, CH), jnp.int32),
        pltpu.VMEM((CAP,), jnp.int32),
        pltpu.VMEM((CAP,), jnp.int32),
        pltpu.VMEM((CAP,), jnp.int32),
        pltpu.VMEM((CAP,), jnp.int32),
        pltpu.VMEM((CH,), jnp.int32),
    ]
    NV = CH // VR

    def body(srcw, dstw, psrc, pdst, tcnt, src_v, dst_v, ls_v, ld_v, hs_v,
             hd_v, cnt_v):
        c = lax.axis_index("c")
        s = lax.axis_index("s")
        w = c * NS + s
        pltpu.sync_copy(srcw.at[w], src_v)
        pltpu.sync_copy(dstw.at[w], dst_v)

        def vstep(i, offs):
            off_lo, off_hi = offs
            j = i // NV
            k = i - j * NV
            sv = src_v[j, pl.ds(k * VR, VR)]
            dv = dst_v[j, pl.ds(k * VR, VR)]
            m_lo = dv < NH
            plsc.store_compressed(ls_v.at[pl.ds(off_lo, VR)], sv, mask=m_lo)
            plsc.store_compressed(ld_v.at[pl.ds(off_lo, VR)], dv, mask=m_lo)
            n_lo = jnp.max(plsc.all_reduce_population_count(m_lo))
            m_hi = jnp.logical_not(m_lo)
            plsc.store_compressed(hs_v.at[pl.ds(off_hi, VR)], sv, mask=m_hi)
            plsc.store_compressed(hd_v.at[pl.ds(off_hi, VR)], dv - NH, mask=m_hi)
            return off_lo + n_lo, off_hi + (VR - n_lo)

        off_lo, off_hi = lax.fori_loop(0, NCHK_D * NV, vstep,
                                       (jnp.int32(0), jnp.int32(0)),
                                       unroll=False)
        zero16 = jnp.zeros((VR,), jnp.int32)
        dum16 = NH + lax.iota(jnp.int32, VR)
        for k in range(NV):
            ls_v[pl.ds(off_lo + k * VR, VR)] = zero16
            ld_v[pl.ds(off_lo + k * VR, VR)] = dum16
            hs_v[pl.ds(off_hi + k * VR, VR)] = zero16
            hd_v[pl.ds(off_hi + k * VR, VR)] = dum16
        t_lo = (off_lo + CH - 1) // CH
        t_hi = (off_hi + CH - 1) // CH
        pltpu.sync_copy(ls_v, psrc.at[0, w])
        pltpu.sync_copy(ld_v, pdst.at[0, w])
        pltpu.sync_copy(hs_v, psrc.at[1, w])
        pltpu.sync_copy(hd_v, pdst.at[1, w])
        for k in range(NV):
            cnt_v[pl.ds(k * VR, VR)] = jnp.zeros((VR,), jnp.int32) + t_lo
        pltpu.sync_copy(cnt_v, tcnt.at[0, w])
        for k in range(NV):
            cnt_v[pl.ds(k * VR, VR)] = jnp.zeros((VR,), jnp.int32) + t_hi
        pltpu.sync_copy(cnt_v, tcnt.at[1, w])

    return pl.kernel(body, out_type=out_type, mesh=mesh,
                     scratch_types=scratch, name="sc_part",
                     compiler_params=pltpu.CompilerParams(
                         needs_layout_passes=False))


def _make_sc_agg2():
    mesh = plsc.VectorSubcoreMesh(core_axis_name="c", subcore_axis_name="s")
    scratch = [
        pltpu.VMEM((2 * CAPC, CH), jnp.int32),
        pltpu.VMEM((2 * CAPC, CH), jnp.int32),
        pltpu.VMEM((CH,), jnp.int32),
        pltpu.VMEM((CH,), jnp.int32),
        pltpu.VMEM((CH,), jnp.int32),
        pltpu.VMEM((CH, D), jnp.float32),
        pltpu.VMEM((CH, D), jnp.float32),
        pltpu.VMEM_SHARED((NPH, D), jnp.float32),
        pltpu.SemaphoreType.DMA,
        pltpu.SemaphoreType.DMA,
    ]

    def body(h_hbm, psrc, pdst, tcnt, zrow, out_p, src_v, dst_v, dloc_v,
             cta_v, ctb_v, rows_a, rows_b, acc_sh, sem_a, sem_b):
        c = lax.axis_index("c")
        s = lax.axis_index("s")
        r0 = s * RPH
        wa = 2 * s
        wb = 2 * s + 1
        pltpu.sync_copy(psrc.at[c, wa], src_v.at[pl.ds(0, CAPC)])
        pltpu.sync_copy(psrc.at[c, wb], src_v.at[pl.ds(CAPC, CAPC)])
        pltpu.sync_copy(pdst.at[c, wa], dst_v.at[pl.ds(0, CAPC)])
        pltpu.sync_copy(pdst.at[c, wb], dst_v.at[pl.ds(CAPC, CAPC)])
        pltpu.sync_copy(tcnt.at[c, wa], cta_v)
        pltpu.sync_copy(tcnt.at[c, wb], ctb_v)
        ta = cta_v[pl.ds(0, VR)][0]
        tb = ctb_v[pl.ds(0, VR)][0]
        T = ta + tb
        pltpu.sync_copy(zrow.at[pl.ds(r0, RPH)], acc_sh.at[pl.ds(r0, RPH)])
        plsc.subcore_barrier()

        def remap(rj):
            for k in range(CH // VR):
                dloc_v[pl.ds(k * VR, VR)] = dst_v[rj, pl.ds(k * VR, VR)]

        def run_list(base, trips):
            @pl.when(trips > 0)
            def _():
                pltpu.async_copy(h_hbm.at[src_v.at[base]], rows_a, sem_a)

                def chunk(j, _):
                    even = lax.rem(j, 2) == 0

                    def do(rows_cur, sem_cur, rows_nxt, sem_nxt):
                        pltpu.async_copy(h_hbm.at[src_v.at[base + j + 1]],
                                         rows_nxt, sem_nxt)
                        remap(base + j)
                        pltpu.make_async_copy(h_hbm.at[src_v.at[base + j]],
                                              rows_cur, sem_cur).wait()
                        pltpu.sync_copy(rows_cur, acc_sh.at[dloc_v], add=True)

                    lax.cond(even,
                             lambda: do(rows_a, sem_a, rows_b, sem_b),
                             lambda: do(rows_b, sem_b, rows_a, sem_a))
                    return 0

                lax.fori_loop(0, trips - 1, chunk, 0, unroll=False)
                jlast = trips - 1

                def last(rows_cur, sem_cur):
                    remap(base + jlast)
                    pltpu.make_async_copy(h_hbm.at[src_v.at[base + jlast]],
                                          rows_cur, sem_cur).wait()
                    pltpu.sync_copy(rows_cur, acc_sh.at[dloc_v], add=True)

                lax.cond(lax.rem(jlast, 2) == 0,
                         lambda: last(rows_a, sem_a),
                         lambda: last(rows_b, sem_b))

        run_list(0, ta)
        run_list(CAPC, tb)

        plsc.subcore_barrier()
        pltpu.sync_copy(acc_sh.at[pl.ds(r0, RPH)], out_p.at[c, pl.ds(r0, RPH)])

    return pl.kernel(body,
                     out_type=jax.ShapeDtypeStruct((NC, NPH, D), jnp.float32),
                     mesh=mesh, scratch_types=scratch, name="sc_segsum2")


def _make_sc_deg():
    mesh = plsc.VectorSubcoreMesh(core_axis_name="c", subcore_axis_name="s")
    scratch = [
        pltpu.VMEM((NCHK_D, CH), jnp.int32),
        pltpu.VMEM((NPAD,), jnp.float32),
    ]

    def body(dstw, zdeg, deg_p, dst_v, deg_v):
        c = lax.axis_index("c")
        s = lax.axis_index("s")
        wid = c * NS + s
        pltpu.sync_copy(dstw.at[wid], dst_v)
        pltpu.sync_copy(zdeg, deg_v)
        ones = jnp.ones((VR,), jnp.float32)

        def chunk(j, _):
            for k in range(CH // VR):
                idx = dst_v[j, pl.ds(k * VR, VR)]
                plsc.addupdate_scatter(deg_v, [idx], ones)
            return 0

        lax.fori_loop(0, NCHK_D, chunk, 0, unroll=False)
        pltpu.sync_copy(deg_v, deg_p.at[wid])

    return pl.kernel(body,
                     out_type=jax.ShapeDtypeStruct((NW, NPAD), jnp.float32),
                     mesh=mesh, scratch_types=scratch, name="sc_deg",
                     compiler_params=pltpu.CompilerParams(
                         needs_layout_passes=False))


_sc_agg = _make_sc_agg()
_sc_part = _make_sc_part()
_sc_agg2 = _make_sc_agg2()
_sc_deg = _make_sc_deg()


RB = 1000
NBH = NH // RB

_SQRT_HALF = 0.7071067811865476


def _gelu(x):
    return 0.5 * x * (1.0 + lax.erf(x * _SQRT_HALF))


RB2 = 1024


def _tc_deginv_body(dg_ref, o_ref):
    dsum = jnp.sum(dg_ref[...], axis=0)
    inv = 1.0 / jnp.maximum(dsum, 1.0)
    o_ref[...] = jnp.broadcast_to(inv, (8, RB2)).T


def _tc_deginv(degs):
    return pl.pallas_call(
        _tc_deginv_body,
        grid=(NPAD // RB2,),
        in_specs=[pl.BlockSpec((NW, RB2), lambda i: (0, i))],
        out_specs=pl.BlockSpec((RB2, 8), lambda i: (i, 0)),
        out_shape=jax.ShapeDtypeStruct((NPAD, 8), jnp.float32),
    )(degs)


def _tc_in_body(x_ref, w_ref, b_ref, o_ref):
    t = jnp.dot(x_ref[...], w_ref[...], preferred_element_type=jnp.float32)
    o_ref[...] = _gelu(t + b_ref[...])


def _tc_in(x, w, b):
    return pl.pallas_call(
        _tc_in_body,
        grid=(N // RB,),
        in_specs=[
            pl.BlockSpec((RB, D), lambda i: (i, 0)),
            pl.BlockSpec((D, D), lambda i: (0, 0)),
            pl.BlockSpec((1, D), lambda i: (0, 0)),
        ],
        out_specs=pl.BlockSpec((RB, D), lambda i: (i, 0)),
        out_shape=jax.ShapeDtypeStruct((N, D), jnp.float32),
    )(x, w, b)


def _tc_layer_body(final, p_ref, dg_ref, h_ref, wl_ref, bl_ref, wr_ref,
                   g_ref, be_ref, *rest):
    if final:
        wo_ref, bo_ref, o_ref = rest
    else:
        (o_ref,) = rest
    agg = p_ref[0]
    mean = agg * dg_ref[:, 0:1]
    t = (jnp.dot(mean, wl_ref[...], preferred_element_type=jnp.float32)
         + jnp.dot(h_ref[...], wr_ref[...], preferred_element_type=jnp.float32)
         + bl_ref[...])
    mu = jnp.mean(t, axis=-1, keepdims=True)
    var = jnp.mean((t - mu) ** 2, axis=-1, keepdims=True)
    y = (t - mu) * lax.rsqrt(var + 1e-5) * g_ref[...] + be_ref[...]
    h_new = _gelu(y) + h_ref[...]
    if final:
        o_ref[...] = (jnp.dot(h_new, wo_ref[...],
                              preferred_element_type=jnp.float32)
                      + bo_ref[...])
    else:
        o_ref[...] = h_new


def _tc_layer(parts, degs, h, wl, bl, wr, g, be, wo=None, bo=None):
    final = wo is not None
    in_specs = [
        pl.BlockSpec((1, RB, D), lambda i: (i // NBH, i % NBH, 0)),
        pl.BlockSpec((RB, 8), lambda i: (i, 0)),
        pl.BlockSpec((RB, D), lambda i: (i, 0)),
        pl.BlockSpec((D, D), lambda i: (0, 0)),
        pl.BlockSpec((1, D), lambda i: (0, 0)),
        pl.BlockSpec((D, D), lambda i: (0, 0)),
        pl.BlockSpec((1, D), lambda i: (0, 0)),
        pl.BlockSpec((1, D), lambda i: (0, 0)),
    ]
    args = [parts, degs, h, wl, bl, wr, g, be]
    if final:
        in_specs += [pl.BlockSpec((D, D), lambda i: (0, 0)),
                     pl.BlockSpec((1, D), lambda i: (0, 0))]
        args += [wo, bo]
    return pl.pallas_call(
        functools.partial(_tc_layer_body, final),
        grid=(N // RB,),
        in_specs=in_specs,
        out_specs=pl.BlockSpec((RB, D), lambda i: (i, 0)),
        out_shape=jax.ShapeDtypeStruct((N, D), jnp.float32),
    )(*args)


def kernel(x, edge_index, W_in, b_in, Wl0, bl0, Wr0, g0, be0, Wl1, bl1, Wr1,
           g1, be1, Wl2, bl2, Wr2, g2, be2, W_out, b_out):
    src, dst = edge_index[0], edge_index[1]
    pad_d = NW * EPW_D - E
    srcw_d = jnp.concatenate(
        [src, jnp.zeros((pad_d,), jnp.int32)]).reshape(NW, NCHK_D, CH)
    dstw_d = jnp.concatenate(
        [dst, jnp.full((pad_d,), N, jnp.int32)]).reshape(NW, NCHK_D, CH)
    zrow = jnp.zeros((NPH, D), jnp.float32)
    zdeg = jnp.zeros((NPAD,), jnp.float32)

    h = _tc_in(x, W_in, b_in.reshape(1, D))
    psrc, pdst, tcnt = _sc_part(srcw_d, dstw_d)
    psrc = psrc.reshape(NC, NW, CAPC, CH)
    pdst = pdst.reshape(NC, NW, CAPC, CH)
    degs = _tc_deginv(_sc_deg(dstw_d, zdeg))
    parts = _sc_agg2(h, psrc, pdst, tcnt, zrow)
    h = _tc_layer(parts, degs, h, Wl0, bl0.reshape(1, D), Wr0,
                  g0.reshape(1, D), be0.reshape(1, D))
    parts = _sc_agg2(h, psrc, pdst, tcnt, zrow)
    h = _tc_layer(parts, degs, h, Wl1, bl1.reshape(1, D), Wr1,
                  g1.reshape(1, D), be1.reshape(1, D))
    parts = _sc_agg2(h, psrc, pdst, tcnt, zrow)
    out = _tc_layer(parts, degs, h, Wl2, bl2.reshape(1, D), Wr2,
                    g2.reshape(1, D), be2.reshape(1, D),
                    W_out, b_out.reshape(1, D))
    return out


# speedup vs baseline: 1.4321x; 1.4321x over previous
import functools

import jax
import jax.numpy as jnp
from jax import lax
from jax.experimental import pallas as pl
from jax.experimental.pallas import tpu as pltpu
from jax.experimental.pallas import tpu_sc as plsc

N, E, D = 10000, 320000, 128
NC, NS = 2, 16
NW = NC * NS
NH = N // NC
CH = 128
VR = 16
EPW_A = ((E // NS + CH - 1) // CH) * CH
NCHK_A = EPW_A // CH
EPW_D = ((E // NW + CH - 1) // CH) * CH
NCHK_D = EPW_D // CH
NPH = 5120
RPH = NPH // NS
NDUM = 48
NPAD = 10240
RPS = NPAD // NS


def _make_sc_agg():
    mesh = plsc.VectorSubcoreMesh(core_axis_name="c", subcore_axis_name="s")
    scratch = [
        pltpu.VMEM((NCHK_A, CH), jnp.int32),
        pltpu.VMEM((NCHK_A, CH), jnp.int32),
        pltpu.VMEM((CH,), jnp.int32),
        pltpu.VMEM((CH,), jnp.int32),
        pltpu.VMEM((CH, D), jnp.float32),
        pltpu.VMEM((CH, D), jnp.float32),
        pltpu.VMEM_SHARED((NPH, D), jnp.float32),
        pltpu.SemaphoreType.DMA,
        pltpu.SemaphoreType.DMA,
        pltpu.SemaphoreType.DMA,
        pltpu.SemaphoreType.DMA,
    ]

    def body(h_hbm, srcw, dstw, zrow, out_p, src_v, dst_v, dl0, dl1,
             rw0, rw1, acc_sh, sg0, sg1, ss0, ss1):
        c = lax.axis_index("c")
        s = lax.axis_index("s")
        r0 = s * RPH
        lo = c * NH
        rows = (rw0, rw1)
        dloc = (dl0, dl1)
        sem_g = (sg0, sg1)
        sem_s = (ss0, ss1)
        pltpu.sync_copy(srcw.at[s], src_v)
        pltpu.sync_copy(dstw.at[s], dst_v)
        pltpu.sync_copy(zrow.at[pl.ds(r0, RPH)], acc_sh.at[pl.ds(r0, RPH)])
        plsc.subcore_barrier()

        def remap(j, dv):
            for k in range(CH // VR):
                d = dst_v[j, pl.ds(k * VR, VR)]
                dl = d - lo
                mine = (dl >= 0) & (dl < NH)
                dummy = NH + (j * (CH // VR) + k) % NDUM
                dv[pl.ds(k * VR, VR)] = jnp.where(mine, dl, dummy)

        def wait_scat(p):
            pltpu.make_async_copy(rows[p], acc_sh.at[dloc[p]],
                                  sem_s[p]).wait()

        pltpu.async_copy(h_hbm.at[src_v.at[0]], rw0, sg0)

        def chunk(j, _):
            def it(p):
                pn = 1 - p
                lax.cond(j >= 1, lambda: wait_scat(pn), lambda: None)
                def issue_next():
                    pltpu.async_copy(h_hbm.at[src_v.at[j + 1]], rows[pn],
                                     sem_g[pn])

                lax.cond(j + 1 < NCHK_A, issue_next, lambda: None)
                remap(j, dloc[p])
                pltpu.make_async_copy(h_hbm.at[src_v.at[j]], rows[p],
                                      sem_g[p]).wait()
                pltpu.async_copy(rows[p], acc_sh.at[dloc[p]], sem_s[p],
                                 add=True)

            lax.cond(lax.rem(j, 2) == 0, lambda: it(0), lambda: it(1))
            return 0

        lax.fori_loop(0, NCHK_A, chunk, 0, unroll=False)
        wait_scat((NCHK_A - 1) % 2)
        plsc.subcore_barrier()
        pltpu.sync_copy(acc_sh.at[pl.ds(r0, RPH)], out_p.at[c, pl.ds(r0, RPH)])

    return pl.kernel(body,
                     out_type=jax.ShapeDtypeStruct((NC, NPH, D), jnp.float32),
                     mesh=mesh, scratch_types=scratch, name="sc_segsum")


def _make_sc_deg():
    mesh = plsc.VectorSubcoreMesh(core_axis_name="c", subcore_axis_name="s")
    scratch = [
        pltpu.VMEM((NCHK_D, CH), jnp.int32),
        pltpu.VMEM((NPAD,), jnp.float32),
    ]

    def body(dstw, zdeg, deg_p, dst_v, deg_v):
        c = lax.axis_index("c")
        s = lax.axis_index("s")
        wid = c * NS + s
        pltpu.sync_copy(dstw.at[wid], dst_v)
        pltpu.sync_copy(zdeg, deg_v)
        ones = jnp.ones((VR,), jnp.float32)

        def chunk(j, _):
            for k in range(CH // VR):
                idx = dst_v[j, pl.ds(k * VR, VR)]
                plsc.addupdate_scatter(deg_v, [idx], ones)
            return 0

        lax.fori_loop(0, NCHK_D, chunk, 0, unroll=False)
        pltpu.sync_copy(deg_v, deg_p.at[wid])

    return pl.kernel(body,
                     out_type=jax.ShapeDtypeStruct((NW, NPAD), jnp.float32),
                     mesh=mesh, scratch_types=scratch, name="sc_deg",
                     compiler_params=pltpu.CompilerParams(
                         needs_layout_passes=False))


_sc_agg = _make_sc_agg()
_sc_deg = _make_sc_deg()


RB = 1000
NBH = NH // RB

_SQRT_HALF = 0.7071067811865476


def _gelu(x):
    return 0.5 * x * (1.0 + lax.erf(x * _SQRT_HALF))


RB2 = 1024


def _tc_deginv_body(dg_ref, o_ref):
    dsum = jnp.sum(dg_ref[...], axis=0)
    inv = 1.0 / jnp.maximum(dsum, 1.0)
    o_ref[...] = jnp.broadcast_to(inv, (8, RB2)).T


def _tc_deginv(degs):
    return pl.pallas_call(
        _tc_deginv_body,
        grid=(NPAD // RB2,),
        in_specs=[pl.BlockSpec((NW, RB2), lambda i: (0, i))],
        out_specs=pl.BlockSpec((RB2, 8), lambda i: (i, 0)),
        out_shape=jax.ShapeDtypeStruct((NPAD, 8), jnp.float32),
    )(degs)


def _tc_in_body(x_ref, w_ref, b_ref, o_ref):
    t = jnp.dot(x_ref[...], w_ref[...], preferred_element_type=jnp.float32)
    o_ref[...] = _gelu(t + b_ref[...])


def _tc_in(x, w, b):
    return pl.pallas_call(
        _tc_in_body,
        grid=(N // RB,),
        in_specs=[
            pl.BlockSpec((RB, D), lambda i: (i, 0)),
            pl.BlockSpec((D, D), lambda i: (0, 0)),
            pl.BlockSpec((1, D), lambda i: (0, 0)),
        ],
        out_specs=pl.BlockSpec((RB, D), lambda i: (i, 0)),
        out_shape=jax.ShapeDtypeStruct((N, D), jnp.float32),
    )(x, w, b)


def _tc_layer_body(final, p_ref, dg_ref, h_ref, wl_ref, bl_ref, wr_ref,
                   g_ref, be_ref, *rest):
    if final:
        wo_ref, bo_ref, o_ref = rest
    else:
        (o_ref,) = rest
    agg = p_ref[0]
    mean = agg * dg_ref[:, 0:1]
    t = (jnp.dot(mean, wl_ref[...], preferred_element_type=jnp.float32)
         + jnp.dot(h_ref[...], wr_ref[...], preferred_element_type=jnp.float32)
         + bl_ref[...])
    mu = jnp.mean(t, axis=-1, keepdims=True)
    var = jnp.mean((t - mu) ** 2, axis=-1, keepdims=True)
    y = (t - mu) * lax.rsqrt(var + 1e-5) * g_ref[...] + be_ref[...]
    h_new = _gelu(y) + h_ref[...]
    if final:
        o_ref[...] = (jnp.dot(h_new, wo_ref[...],
                              preferred_element_type=jnp.float32)
                      + bo_ref[...])
    else:
        o_ref[...] = h_new


def _tc_layer(parts, degs, h, wl, bl, wr, g, be, wo=None, bo=None):
    final = wo is not None
    in_specs = [
        pl.BlockSpec((1, RB, D), lambda i: (i // NBH, i % NBH, 0)),
        pl.BlockSpec((RB, 8), lambda i: (i, 0)),
        pl.BlockSpec((RB, D), lambda i: (i, 0)),
        pl.BlockSpec((D, D), lambda i: (0, 0)),
        pl.BlockSpec((1, D), lambda i: (0, 0)),
        pl.BlockSpec((D, D), lambda i: (0, 0)),
        pl.BlockSpec((1, D), lambda i: (0, 0)),
        pl.BlockSpec((1, D), lambda i: (0, 0)),
    ]
    args = [parts, degs, h, wl, bl, wr, g, be]
    if final:
        in_specs += [pl.BlockSpec((D, D), lambda i: (0, 0)),
                     pl.BlockSpec((1, D), lambda i: (0, 0))]
        args += [wo, bo]
    return pl.pallas_call(
        functools.partial(_tc_layer_body, final),
        grid=(N // RB,),
        in_specs=in_specs,
        out_specs=pl.BlockSpec((RB, D), lambda i: (i, 0)),
        out_shape=jax.ShapeDtypeStruct((N, D), jnp.float32),
    )(*args)


def kernel(x, edge_index, W_in, b_in, Wl0, bl0, Wr0, g0, be0, Wl1, bl1, Wr1,
           g1, be1, Wl2, bl2, Wr2, g2, be2, W_out, b_out):
    src, dst = edge_index[0], edge_index[1]
    pad_a = NS * EPW_A - E
    srcw = jnp.concatenate(
        [src, jnp.zeros((pad_a,), jnp.int32)]).reshape(NS, NCHK_A, CH)
    dstw = jnp.concatenate(
        [dst, jnp.full((pad_a,), -1, jnp.int32)]).reshape(NS, NCHK_A, CH)
    pad_d = NW * EPW_D - E
    dstw_d = jnp.concatenate(
        [dst, jnp.full((pad_d,), N, jnp.int32)]).reshape(NW, NCHK_D, CH)
    zrow = jnp.zeros((NPH, D), jnp.float32)
    zdeg = jnp.zeros((NPAD,), jnp.float32)

    h = _tc_in(x, W_in, b_in.reshape(1, D))
    degs = _tc_deginv(_sc_deg(dstw_d, zdeg))
    parts = _sc_agg(h, srcw, dstw, zrow)
    h = _tc_layer(parts, degs, h, Wl0, bl0.reshape(1, D), Wr0,
                  g0.reshape(1, D), be0.reshape(1, D))
    parts = _sc_agg(h, srcw, dstw, zrow)
    h = _tc_layer(parts, degs, h, Wl1, bl1.reshape(1, D), Wr1,
                  g1.reshape(1, D), be1.reshape(1, D))
    parts = _sc_agg(h, srcw, dstw, zrow)
    out = _tc_layer(parts, degs, h, Wl2, bl2.reshape(1, D), Wr2,
                    g2.reshape(1, D), be2.reshape(1, D),
                    W_out, b_out.reshape(1, D))
    return out
